# Initial kernel scaffold; baseline (speedup 1.0000x reference)
import functools

import jax
import jax.numpy as jnp
from jax import lax
from jax.experimental import pallas as pl
from jax.experimental.pallas import tpu as pltpu
from jax.experimental.pallas import tpu_sc as plsc

DIM = 64
CHUNK = 128
LANES = 16
NC, NS = 2, 16
NW = NC * NS
GROUPS = CHUNK // LANES


def _rsqrt(s):
    i = plsc.bitcast(s, jnp.int32)
    i = jnp.int32(0x5F3759DF) - lax.shift_right_logical(i, 1)
    y = plsc.bitcast(i, jnp.float32)
    for _ in range(3):
        y = y * (1.5 - 0.5 * s * y * y)
    return y


def _normalize_group(in_v, out_v, g, carry):
    rows = g * LANES + lax.iota(jnp.int32, LANES)
    acc = jnp.zeros((LANES,), jnp.float32)
    for j in range(DIM):
        col = jnp.full((LANES,), j, jnp.int32)
        v = plsc.load_gather(in_v, [rows, col])
        acc = acc + v * v
    y = _rsqrt(acc)
    for j in range(DIM):
        col = jnp.full((LANES,), j, jnp.int32)
        v = plsc.load_gather(in_v, [rows, col])
        plsc.store_scatter(out_v, [rows, col], v * y)
    return carry


def _make_lookup(n_rows):
    steps = n_rows // (NW * CHUNK)
    mesh = plsc.VectorSubcoreMesh(core_axis_name="c", subcore_axis_name="s")

    @functools.partial(
        pl.kernel,
        mesh=mesh,
        out_type=jax.ShapeDtypeStruct((n_rows, DIM), jnp.float32),
        scratch_types=[
            pltpu.VMEM((steps, CHUNK), jnp.int32),
            pltpu.VMEM((CHUNK, DIM), jnp.float32),
            pltpu.VMEM((CHUNK, DIM), jnp.float32),
            pltpu.SemaphoreType.DMA,
        ],
    )
    def lookup(x_hbm, tbl_hbm, out_hbm, idx_v, in_v, out_v, sem):
        w = lax.axis_index("s") * NC + lax.axis_index("c")
        pltpu.sync_copy(x_hbm.at[pl.ds(w * steps, steps)], idx_v)

        def step(s, carry):
            pltpu.async_copy(tbl_hbm.at[idx_v.at[s]], in_v, sem).wait()
            lax.fori_loop(
                0, GROUPS, functools.partial(_normalize_group, in_v, out_v), 0
            )
            base = (w * steps + s) * CHUNK
            pltpu.sync_copy(out_v, out_hbm.at[pl.ds(base, CHUNK)])
            return carry

        lax.fori_loop(0, steps, step, 0)

    return lookup


def kernel(x, W_inner):
    b, l = x.shape
    n = b * l
    xi = x.astype(jnp.int32).reshape(n // CHUNK, CHUNK)
    out = _make_lookup(n)(xi, W_inner)
    return out.reshape(b, l, DIM)


# speedup vs baseline: 1.5976x; 1.5976x over previous
import functools

import jax
import jax.numpy as jnp
from jax import lax
from jax.experimental import pallas as pl
from jax.experimental.pallas import tpu as pltpu
from jax.experimental.pallas import tpu_sc as plsc

DIM = 64
CHUNK = 128
LANES = 16
NC, NS = 2, 16
NW = NC * NS
GROUPS = CHUNK // LANES


def _rsqrt(s):
    i = plsc.bitcast(s, jnp.int32)
    i = jnp.int32(0x5F3759DF) - lax.shift_right_logical(i, 1)
    y = plsc.bitcast(i, jnp.float32)
    for _ in range(3):
        y = y * (1.5 - 0.5 * s * y * y)
    return y


def _shuffle(x, idx):
    dn = lax.GatherDimensionNumbers(
        offset_dims=(), collapsed_slice_dims=(0,), start_index_map=(0,)
    )
    return lax.gather(
        x, idx[:, None], dn, (1,),
        mode=lax.GatherScatterMode.PROMISE_IN_BOUNDS,
    )


def _hsum(acc):
    lanes = lax.iota(jnp.int32, LANES)
    for sh in (8, 4, 2, 1):
        perm = jnp.bitwise_xor(lanes, sh)
        acc = acc + _shuffle(acc, perm)
    return acc


def _normalize_row(in_v, out_v, r, carry):
    vs = [in_v[r, pl.ds(k * LANES, LANES)] for k in range(DIM // LANES)]
    acc = vs[0] * vs[0]
    for v in vs[1:]:
        acc = acc + v * v
    y = _rsqrt(_hsum(acc))
    for k, v in enumerate(vs):
        out_v[r, pl.ds(k * LANES, LANES)] = v * y
    return carry


def _make_lookup(n_rows):
    steps = n_rows // (NW * CHUNK)
    mesh = plsc.VectorSubcoreMesh(core_axis_name="c", subcore_axis_name="s")

    @functools.partial(
        pl.kernel,
        mesh=mesh,
        compiler_params=pltpu.CompilerParams(
            needs_layout_passes=False, use_tc_tiling_on_sc=False
        ),
        out_type=jax.ShapeDtypeStruct((n_rows, DIM), jnp.float32),
        scratch_types=[
            pltpu.VMEM((steps, CHUNK), jnp.int32),
            pltpu.VMEM((CHUNK, DIM), jnp.float32),
            pltpu.VMEM((CHUNK, DIM), jnp.float32),
            pltpu.SemaphoreType.DMA,
        ],
    )
    def lookup(x_hbm, tbl_hbm, out_hbm, idx_v, in_v, out_v, sem):
        w = lax.axis_index("s") * NC + lax.axis_index("c")
        pltpu.sync_copy(x_hbm.at[pl.ds(w * steps, steps)], idx_v)

        def step(s, carry):
            pltpu.async_copy(tbl_hbm.at[idx_v.at[s]], in_v, sem).wait()
            lax.fori_loop(
                0, CHUNK, functools.partial(_normalize_row, in_v, out_v), 0
            )
            base = (w * steps + s) * CHUNK
            pltpu.sync_copy(out_v, out_hbm.at[pl.ds(base, CHUNK)])
            return carry

        lax.fori_loop(0, steps, step, 0)

    return lookup


def kernel(x, W_inner):
    b, l = x.shape
    n = b * l
    xi = x.astype(jnp.int32).reshape(n // CHUNK, CHUNK)
    out = _make_lookup(n)(xi, W_inner)
    return out.reshape(b, l, DIM)


# speedup vs baseline: 1.9218x; 1.2030x over previous
import functools

import jax
import jax.numpy as jnp
from jax import lax
from jax.experimental import pallas as pl
from jax.experimental.pallas import tpu as pltpu
from jax.experimental.pallas import tpu_sc as plsc

DIM = 64
CHUNK = 128
LANES = 16
NC, NS = 2, 16
NW = NC * NS
GROUPS = CHUNK // LANES


def _rsqrt(s):
    i = plsc.bitcast(s, jnp.int32)
    i = jnp.int32(0x5F3759DF) - lax.shift_right_logical(i, 1)
    y = plsc.bitcast(i, jnp.float32)
    for _ in range(3):
        y = y * (1.5 - 0.5 * s * y * y)
    return y


def _shuffle(x, idx):
    dn = lax.GatherDimensionNumbers(
        offset_dims=(), collapsed_slice_dims=(0,), start_index_map=(0,)
    )
    return lax.gather(
        x, idx[:, None], dn, (1,),
        mode=lax.GatherScatterMode.PROMISE_IN_BOUNDS,
    )


def _hsum(acc):
    lanes = lax.iota(jnp.int32, LANES)
    for sh in (8, 4, 2, 1):
        perm = jnp.bitwise_xor(lanes, sh)
        acc = acc + _shuffle(acc, perm)
    return acc


def _normalize_row(in_v, out_v, r, carry):
    vs = [in_v[r, pl.ds(k * LANES, LANES)] for k in range(DIM // LANES)]
    acc = vs[0] * vs[0]
    for v in vs[1:]:
        acc = acc + v * v
    y = _rsqrt(_hsum(acc))
    for k, v in enumerate(vs):
        out_v[r, pl.ds(k * LANES, LANES)] = v * y
    return carry


NBUF = 4


def _make_lookup(n_rows):
    steps = n_rows // (NW * CHUNK)
    mesh = plsc.VectorSubcoreMesh(core_axis_name="c", subcore_axis_name="s")

    @functools.partial(
        pl.kernel,
        mesh=mesh,
        compiler_params=pltpu.CompilerParams(
            needs_layout_passes=False, use_tc_tiling_on_sc=False
        ),
        out_type=jax.ShapeDtypeStruct((n_rows, DIM), jnp.float32),
        scratch_types=[
            pltpu.VMEM((steps, CHUNK), jnp.int32),
            pltpu.VMEM((NBUF, CHUNK, DIM), jnp.float32),
            pltpu.VMEM((NBUF, CHUNK, DIM), jnp.float32),
            pltpu.SemaphoreType.DMA((NBUF,)),
            pltpu.SemaphoreType.DMA((NBUF,)),
        ],
    )
    def lookup(x_hbm, tbl_hbm, out_hbm, idx_v, in_v, out_v, sem_g, sem_s):
        w = lax.axis_index("s") * NC + lax.axis_index("c")
        pltpu.sync_copy(x_hbm.at[pl.ds(w * steps, steps)], idx_v)

        def gather(s, b):
            return pltpu.make_async_copy(
                tbl_hbm.at[idx_v.at[s]], in_v.at[b], sem_g.at[b]
            )

        def scatter(s, b):
            base = (w * steps + s) * CHUNK
            return pltpu.make_async_copy(
                out_v.at[b], out_hbm.at[pl.ds(base, CHUNK)], sem_s.at[b]
            )

        for b in range(NBUF):
            gather(b, b).start()

        def round_(t, carry):
            for b in range(NBUF):
                s = t * NBUF + b

                @pl.when(s >= NBUF)
                def _():
                    scatter(s - NBUF, b).wait()

                gather(s, b).wait()
                lax.fori_loop(
                    0,
                    CHUNK,
                    functools.partial(_normalize_row, in_v.at[b], out_v.at[b]),
                    0,
                )
                scatter(s, b).start()

                @pl.when(s + NBUF < steps)
                def _():
                    gather(s + NBUF, b).start()

            return carry

        lax.fori_loop(0, steps // NBUF, round_, 0)
        for b in range(NBUF):
            scatter(steps - NBUF + b, b).wait()

    return lookup


def kernel(x, W_inner):
    b, l = x.shape
    n = b * l
    xi = x.astype(jnp.int32).reshape(n // CHUNK, CHUNK)
    out = _make_lookup(n)(xi, W_inner)
    return out.reshape(b, l, DIM)
